# Initial kernel scaffold; baseline (speedup 1.0000x reference)
import jax
import jax.numpy as jnp
from jax.experimental import pallas as pl


def kernel(fts, adjs_norm, fts_shuf, W_pre, b_pre, a_rel, W_hier, W_hgcn, W_disc):
    raise NotImplementedError("write your pallas kernel here")


# speedup vs baseline: 6.2313x; 6.2313x over previous
import functools

import jax
import jax.numpy as jnp
from jax.experimental import pallas as pl
from jax.experimental.pallas import tpu as pltpu


def _pick_block(n, pref):
    b = min(pref, n)
    while n % b or b % 8:
        b -= 8 if b > 8 else 1
        if b < 8:
            return n
    return b


def _k1_body(x_ref, xs_ref, w_ref, b_ref, o_ref):
    w = w_ref[...]
    b = b_ref[0:1, :]
    h1 = jnp.dot(x_ref[...], w, preferred_element_type=jnp.float32) + b
    h2 = jnp.dot(xs_ref[...], w, preferred_element_type=jnp.float32) + b
    o_ref[...] = jnp.concatenate([h1, h2], axis=1)


def _k2_body(a_ref, h_ref, hid_ref, cs_ref, *, n_rel, inv_rel):
    i = pl.program_id(0)
    r = pl.program_id(1)
    t = jax.nn.relu(jnp.dot(a_ref[0], h_ref[...],
                            preferred_element_type=jnp.float32)) * inv_rel

    @pl.when(r == 0)
    def _init():
        hid_ref[...] = t

    @pl.when(r != 0)
    def _acc():
        hid_ref[...] += t

    @pl.when(r == n_rel - 1)
    def _fin():
        @pl.when(i == 0)
        def _zero():
            cs_ref[...] = jnp.zeros_like(cs_ref)

        cs_ref[0:1, :] += jnp.sum(hid_ref[...], axis=0, keepdims=True)


def _k3_body(h_ref, w1_ref, w2_ref, o_ref):
    zh = jax.nn.relu(jnp.dot(h_ref[...], w1_ref[...],
                             preferred_element_type=jnp.float32))
    o_ref[...] = jnp.dot(zh, w2_ref[...], preferred_element_type=jnp.float32)


def _k4_body(a_ref, y_ref, beta_ref, z_ref, cs_ref, *, n_rel):
    i = pl.program_id(0)
    r = pl.program_id(1)
    t = jnp.dot(a_ref[0], y_ref[...],
                preferred_element_type=jnp.float32) * beta_ref[pl.ds(r, 1), :]

    @pl.when(r == 0)
    def _init():
        z_ref[...] = t

    @pl.when(r != 0)
    def _acc():
        z_ref[...] += t

    @pl.when(r == n_rel - 1)
    def _fin():
        z_ref[...] = jax.nn.relu(z_ref[...])

        @pl.when(i == 0)
        def _zero():
            cs_ref[...] = jnp.zeros_like(cs_ref)

        cs_ref[0:1, :] += jnp.sum(z_ref[...], axis=0, keepdims=True)


def _k5_body(z_ref, wd_ref, s_ref, o_ref, *, d):
    wd = wd_ref[...]
    s = s_ref[0:1, :]
    zp = z_ref[:, :d]
    zn = z_ref[:, d:]
    tp = jnp.dot(zp, wd, preferred_element_type=jnp.float32)
    tn = jnp.dot(zn, wd, preferred_element_type=jnp.float32)
    scp = jnp.sum(tp * s, axis=1, keepdims=True)
    scn = jnp.sum(tn * s, axis=1, keepdims=True)
    o_ref[...] = jnp.concatenate([scp, scn], axis=1)


def kernel(fts, adjs_norm, fts_shuf, W_pre, b_pre, a_rel, W_hier, W_hgcn, W_disc):
    n_rel, n, _ = adjs_norm.shape
    f = fts.shape[-1]
    d = W_pre.shape[-1]
    d2 = 2 * d

    x = fts[0]
    xs = fts_shuf[0]
    b2 = b_pre.reshape(1, d)

    rb1 = _pick_block(n, 2000)
    hcat = pl.pallas_call(
        _k1_body,
        grid=(n // rb1,),
        in_specs=[
            pl.BlockSpec((rb1, f), lambda i: (i, 0)),
            pl.BlockSpec((rb1, f), lambda i: (i, 0)),
            pl.BlockSpec((f, d), lambda i: (0, 0)),
            pl.BlockSpec((1, d), lambda i: (0, 0)),
        ],
        out_specs=pl.BlockSpec((rb1, d2), lambda i: (i, 0)),
        out_shape=jax.ShapeDtypeStruct((n, d2), jnp.float32),
    )(x, xs, W_pre, b2)

    rb = _pick_block(n, 200)
    nr_g = n // rb

    hid, cs_hid = pl.pallas_call(
        functools.partial(_k2_body, n_rel=n_rel, inv_rel=1.0 / n_rel),
        grid=(nr_g, n_rel),
        in_specs=[
            pl.BlockSpec((1, rb, n), lambda i, r: (r, i, 0)),
            pl.BlockSpec((n, d2), lambda i, r: (0, 0)),
        ],
        out_specs=[
            pl.BlockSpec((rb, d2), lambda i, r: (i, 0)),
            pl.BlockSpec((8, d2), lambda i, r: (0, 0)),
        ],
        out_shape=[
            jax.ShapeDtypeStruct((n, d2), jnp.float32),
            jax.ShapeDtypeStruct((8, d2), jnp.float32),
        ],
    )(adjs_norm, hcat)

    cs = cs_hid[0]
    scores_p = a_rel @ cs[:d] / n
    scores_n = a_rel @ cs[d:] / n
    beta_p = jax.nn.softmax(scores_p)
    beta_n = jax.nn.softmax(scores_n)
    beta2 = jnp.zeros((8, d2), jnp.float32)
    beta2 = beta2.at[:n_rel, :d].set(beta_p[:, None])
    beta2 = beta2.at[:n_rel, d:].set(beta_n[:, None])

    zblk = jnp.zeros((d, d), jnp.float32)
    w1bd = jnp.block([[W_hier, zblk], [zblk, W_hier]])
    w2bd = jnp.block([[W_hgcn, zblk], [zblk, W_hgcn]])
    y = pl.pallas_call(
        _k3_body,
        grid=(n // rb1,),
        in_specs=[
            pl.BlockSpec((rb1, d2), lambda i: (i, 0)),
            pl.BlockSpec((d2, d2), lambda i: (0, 0)),
            pl.BlockSpec((d2, d2), lambda i: (0, 0)),
        ],
        out_specs=pl.BlockSpec((rb1, d2), lambda i: (i, 0)),
        out_shape=jax.ShapeDtypeStruct((n, d2), jnp.float32),
    )(hid, w1bd, w2bd)

    z, cs_z = pl.pallas_call(
        functools.partial(_k4_body, n_rel=n_rel),
        grid=(nr_g, n_rel),
        in_specs=[
            pl.BlockSpec((1, rb, n), lambda i, r: (r, i, 0)),
            pl.BlockSpec((n, d2), lambda i, r: (0, 0)),
            pl.BlockSpec((8, d2), lambda i, r: (0, 0)),
        ],
        out_specs=[
            pl.BlockSpec((rb, d2), lambda i, r: (i, 0)),
            pl.BlockSpec((8, d2), lambda i, r: (0, 0)),
        ],
        out_shape=[
            jax.ShapeDtypeStruct((n, d2), jnp.float32),
            jax.ShapeDtypeStruct((8, d2), jnp.float32),
        ],
    )(adjs_norm, y, beta2)

    s = jax.nn.sigmoid(cs_z[0, :d] / n).reshape(1, d)

    sc = pl.pallas_call(
        functools.partial(_k5_body, d=d),
        grid=(n // rb1,),
        in_specs=[
            pl.BlockSpec((rb1, d2), lambda i: (i, 0)),
            pl.BlockSpec((d, d), lambda i: (0, 0)),
            pl.BlockSpec((1, d), lambda i: (0, 0)),
        ],
        out_specs=pl.BlockSpec((rb1, 2), lambda i: (i, 0)),
        out_shape=jax.ShapeDtypeStruct((n, 2), jnp.float32),
    )(z, W_disc, s)

    logits = jnp.concatenate([sc[:, 0], sc[:, 1]]).reshape(1, 2 * n)
    return logits
